# baseline (speedup 1.0000x reference)
import jax
import jax.numpy as jnp
from jax.experimental import pallas as pl

_F = 2048
_T = 32
_H1 = 512
_H2 = 128
_B = 160
_R = 5120


def _clf(x, W1, b1, W2, b2, W3, b3):
    h = jnp.maximum(jnp.dot(x, W1, preferred_element_type=jnp.float32) + b1, 0.0)
    h = jnp.maximum(jnp.dot(h, W2, preferred_element_type=jnp.float32) + b2, 0.0)
    z = jnp.dot(h, W3, preferred_element_type=jnp.float32) + b3
    return jax.nn.sigmoid(z)


def _ref_body(rgb_ref, flow_ref, pW1, pb1, pW2, pb2, pW3, pb3,
              fW1, fb1, fW2, fb2, fW3, fb3, fused_o, ps_o, fs_o):
    rgb = rgb_ref[...]
    flow = flow_ref[...]
    ps_o[...] = _clf(rgb, pW1[...], pb1[...], pW2[...], pb2[...], pW3[...], pb3[...])
    fused = rgb * jax.nn.sigmoid(flow)
    fused_o[...] = fused
    fs_o[...] = _clf(fused, fW1[...], fb1[...], fW2[...], fb2[...], fW3[...], fb3[...])


def _clf_body(x_ref, W1, b1, W2, b2, W3, b3, s_o):
    s_o[...] = _clf(x_ref[...], W1[...], b1[...], W2[...], b2[...], W3[...], b3[...])


def _topk_onehots(v, k):
    n = v.shape[1]
    iota = jax.lax.broadcasted_iota(jnp.int32, (1, n), 1)
    rows = []
    cur = v
    for _ in range(k):
        m = jnp.max(cur, axis=1, keepdims=True)
        idx = jnp.min(jnp.where(cur == m, iota, n), axis=1, keepdims=True)
        oh = iota == idx
        rows.append(oh.astype(jnp.float32))
        cur = jnp.where(oh, -jnp.inf, cur)
    return jnp.concatenate(rows, axis=0)


def _select_body(sr_ref, sf_ref, rgb_ref, flow_ref, sup_o, exf_o):
    sr = sr_ref[...].reshape(1, _T)
    sf = sf_ref[...].reshape(1, _T)
    rgb = rgb_ref[...].reshape(_T, _F)
    flow = flow_ref[...].reshape(_T, _F)

    H = _topk_onehots(-jnp.abs(sr - 0.5), 10)

    iota = jax.lax.broadcasted_iota(jnp.int32, (1, _T), 1)
    hs = jnp.where(iota < 10, 0.0, -jnp.inf)
    hs_neg = hs
    for k in range(10):
        fsk = jnp.sum(H[k:k + 1, :] * sf, axis=1, keepdims=True)
        onek = (iota == k).astype(jnp.float32)
        hs = hs + fsk * onek
        hs_neg = hs_neg - fsk * onek

    Hpad = jnp.concatenate([H, jnp.zeros((_T - 10, _T), jnp.float32)], axis=0)
    j_abn = _topk_onehots(hs, 1)
    j_nor = _topk_onehots(hs_neg, 1)
    t_abn = jnp.dot(j_abn, Hpad, preferred_element_type=jnp.float32)
    t_nor = jnp.dot(j_nor, Hpad, preferred_element_type=jnp.float32)

    Hn = _topk_onehots(-sr, 2)
    Ha = _topk_onehots(sr, 10)

    S1 = jnp.concatenate([Ha, t_abn], axis=0)
    S2 = jnp.concatenate([t_nor, Hn], axis=0)

    sup = jnp.dot(S1, rgb, preferred_element_type=jnp.float32)
    er = jnp.dot(S2, rgb, preferred_element_type=jnp.float32)
    ef = jnp.dot(S2, flow, preferred_element_type=jnp.float32)

    sup_o[...] = sup.reshape(1, 11, _F)
    exf_o[...] = (er * jax.nn.sigmoid(ef)).reshape(1, 3, _F)


def _nor_body(rgb_ref, flow_ref, ex_ref, out_ref):
    out_ref[:, :_T, :] = rgb_ref[...] * jax.nn.sigmoid(flow_ref[...])
    out_ref[:, _T:, :] = ex_ref[...]


def _mean_body(ps_ref, fs_ref, m1_o, m2_o):
    m1_o[...] = jnp.mean(ps_ref[...], axis=1)
    m2_o[...] = jnp.mean(fs_ref[...], axis=1, keepdims=True)


def _wspecs(shapes):
    return [pl.BlockSpec(s, lambda i: (0,) * len(s)) for s in shapes]


_CLF_W_SHAPES = [(_F, _H1), (1, _H1), (_H1, _H2), (1, _H2), (_H2, 1), (1, 1)]


def kernel(ref_rgb, ref_flow, normal_rgb, normal_flow, abnormal_rgb,
           abnormal_flow, prgb_W1, prgb_b1, prgb_W2, prgb_b2, prgb_W3,
           prgb_b3, pflow_W1, pflow_b1, pflow_W2, pflow_b2, pflow_W3,
           pflow_b3, fcls_W1, fcls_b1, fcls_W2, fcls_b2, fcls_W3, fcls_b3):
    f32 = jnp.float32
    prgb = (prgb_W1, prgb_b1.reshape(1, _H1), prgb_W2, prgb_b2.reshape(1, _H2),
            prgb_W3, prgb_b3.reshape(1, 1))
    pflow = (pflow_W1, pflow_b1.reshape(1, _H1), pflow_W2,
             pflow_b2.reshape(1, _H2), pflow_W3, pflow_b3.reshape(1, 1))
    fcls = (fcls_W1, fcls_b1.reshape(1, _H1), fcls_W2, fcls_b2.reshape(1, _H2),
            fcls_W3, fcls_b3.reshape(1, 1))

    r_rgb = ref_rgb.reshape(_R, _F)
    r_flow = ref_flow.reshape(_R, _F)
    a_rgb = abnormal_rgb.reshape(_R, _F)
    a_flow = abnormal_flow.reshape(_R, _F)

    MR = 512
    xspec = pl.BlockSpec((MR, _F), lambda i: (i, 0))
    sspec = pl.BlockSpec((MR, 1), lambda i: (i, 0))
    wsp = _wspecs(_CLF_W_SHAPES)

    fused_ref_flat, ps_ref, fs_ref = pl.pallas_call(
        _ref_body,
        grid=(_R // MR,),
        in_specs=[xspec, xspec] + wsp + wsp,
        out_specs=[xspec, sspec, sspec],
        out_shape=[jax.ShapeDtypeStruct((_R, _F), f32),
                   jax.ShapeDtypeStruct((_R, 1), f32),
                   jax.ShapeDtypeStruct((_R, 1), f32)],
    )(r_rgb, r_flow, *prgb, *fcls)

    clf_call = pl.pallas_call(
        _clf_body,
        grid=(_R // MR,),
        in_specs=[xspec] + wsp,
        out_specs=sspec,
        out_shape=jax.ShapeDtypeStruct((_R, 1), f32),
    )
    s_argb = clf_call(a_rgb, *prgb)
    s_aflow = clf_call(a_flow, *pflow)

    sup, extras = pl.pallas_call(
        _select_body,
        grid=(_B,),
        in_specs=[pl.BlockSpec((1, 1, _T), lambda g: (g, 0, 0)),
                  pl.BlockSpec((1, 1, _T), lambda g: (g, 0, 0)),
                  pl.BlockSpec((1, _T, _F), lambda g: (g, 0, 0)),
                  pl.BlockSpec((1, _T, _F), lambda g: (g, 0, 0))],
        out_specs=[pl.BlockSpec((1, 11, _F), lambda g: (g, 0, 0)),
                   pl.BlockSpec((1, 3, _F), lambda g: (g, 0, 0))],
        out_shape=[jax.ShapeDtypeStruct((_B, 11, _F), f32),
                   jax.ShapeDtypeStruct((_B, 3, _F), f32)],
    )(s_argb.reshape(_B, 1, _T), s_aflow.reshape(_B, 1, _T),
      abnormal_rgb.reshape(_B, _T, _F), abnormal_flow.reshape(_B, _T, _F))

    GB = 4
    fused_nor = pl.pallas_call(
        _nor_body,
        grid=(_B // GB,),
        in_specs=[pl.BlockSpec((GB, _T, _F), lambda g: (g, 0, 0)),
                  pl.BlockSpec((GB, _T, _F), lambda g: (g, 0, 0)),
                  pl.BlockSpec((GB, 3, _F), lambda g: (g, 0, 0))],
        out_specs=pl.BlockSpec((GB, _T + 3, _F), lambda g: (g, 0, 0)),
        out_shape=jax.ShapeDtypeStruct((_B, _T + 3, _F), f32),
    )(normal_rgb.reshape(_B, _T, _F), normal_flow.reshape(_B, _T, _F), extras)

    m1, m2 = pl.pallas_call(
        _mean_body,
        grid=(1,),
        in_specs=[pl.BlockSpec((16, 10, _T), lambda i: (0, 0, 0)),
                  pl.BlockSpec((512, 10), lambda i: (0, 0))],
        out_specs=[pl.BlockSpec((16, _T), lambda i: (0, 0)),
                   pl.BlockSpec((512, 1), lambda i: (0, 0))],
        out_shape=[jax.ShapeDtypeStruct((16, _T), f32),
                   jax.ShapeDtypeStruct((512, 1), f32)],
    )(ps_ref.reshape(16, 10, _T), fs_ref.reshape(512, 10))

    return (m1, m2.reshape(512),
            fused_ref_flat.reshape(_B, _T, _F), fused_nor, sup)
